# baseline (speedup 1.0000x reference)
import functools

import jax
import jax.numpy as jnp
from jax import lax
from jax.experimental import pallas as pl
from jax.experimental.pallas import tpu as pltpu
from jax.experimental.pallas import tpu_sc as plsc

N_USERS = 50000
N_ITEMS = 50000
N = N_USERS + N_ITEMS
EMB = 32
H = 16
N_LAYERS = 3
NT = N_LAYERS + 1
B = 1024
NEG = 64
BN = B * NEG
TAU1 = 0.1
TAU2 = 0.1
W_LAMBDA = 0.5
DECAY = 1e-4
E = 800000

NC, NS, L = 2, 16, 16
NW = NC * NS

N_PAD = 102400
RPT = N_PAD // NS
ECH = 128
SB = 8
NBLK = 50
NIT = NBLK // 2
EPT = NBLK * SB * ECH
E_PAD = EPT * NS
SROWS = E_PAD // ECH
SROWS_P = SROWS + 2 * SB
NBUF = 8
LA = 5
ZR = 64
UPT = B // NS
NPT = BN // NS
PPW = B // NW
NPW = BN // NW

_f32 = jnp.float32
_i32 = jnp.int32


def _sc_body(emb0, e_s, e_d, e_v, u_idx, p_idx, n_idx, up_idx, pp_idx,
             np_idx, pop_u, pop_i,
             layers, u_g, p_g, n_g, up_g, pp_g, np_g,
             sblk, dblk, vblk, rows4, zbuf, r64q, g32, r32, g128, rows32,
             acc, g0, g1, g2, g3, g4, g5, g6, g7,
             s0, s1, s2, s3, s4, s5, s6, s7, semA, semB):
    c = lax.axis_index("c")
    s = lax.axis_index("s")
    w = s * NC + c
    row0 = s * RPT
    gsems = [g0, g1, g2, g3, g4, g5, g6, g7]
    ssems = [s0, s1, s2, s3, s4, s5, s6, s7]

    def _drain_row(b, sem):
        pltpu.make_async_copy(
            emb0.at[0, pl.ds(0, ECH)], rows4.at[b], sem).wait()

    def _stage(slot, r, sem):
        pltpu.async_copy(e_s.at[pl.ds(r, SB)], sblk.at[slot], sem)
        pltpu.async_copy(e_d.at[pl.ds(r, SB)], dblk.at[slot], sem)
        pltpu.async_copy(e_v.at[pl.ds(r, SB)], vblk.at[slot], sem)

    def _drain_stage(slot, sem):
        pltpu.make_async_copy(e_s.at[pl.ds(0, SB)], sblk.at[slot], sem).wait()
        pltpu.make_async_copy(e_d.at[pl.ds(0, SB)], dblk.at[slot], sem).wait()
        pltpu.make_async_copy(e_v.at[pl.ds(0, SB)], vblk.at[slot], sem).wait()

    @pl.loop(0, ZR)
    def _zfill(i):
        zbuf[i] = jnp.zeros((L,), _f32)

    @pl.loop(0, RPT // ZR)
    def _zero0(zz):
        pltpu.sync_copy(zbuf, acc.at[pl.ds(row0 + zz * ZR, ZR)])
    plsc.subcore_barrier()

    r0t = s * (NBLK * SB)

    for k in range(N_LAYERS):
        tab = emb0.at[c] if k == 0 else layers.at[k - 1, c]

        pltpu.sync_copy(e_s.at[pl.ds(r0t, SB)], sblk.at[0])
        pltpu.sync_copy(e_d.at[pl.ds(r0t, SB)], dblk.at[0])
        pltpu.sync_copy(e_v.at[pl.ds(r0t, SB)], vblk.at[0])
        _stage(1, r0t + SB, semB)

        @pl.loop(0, ECH)
        def _zb(i):
            for b in range(LA, NBUF):
                rows4[b, i] = jnp.zeros((L,), _f32)
        for b in range(LA, NBUF):
            pltpu.async_copy(rows4.at[b], acc.at[dblk.at[0, 0]],
                             ssems[b], add=True)
        for b in range(LA):
            pltpu.async_copy(tab.at[sblk.at[0, b]], rows4.at[b],
                             gsems[b])

        @pl.loop(0, NIT)
        def _edge_iter(q):
            cps = [None] * 16
            for jj in range(16):
                slot, row, b = jj // SB, jj % SB, jj % NBUF
                if jj == 5:
                    _drain_stage(1, semB)
                if jj == 8:
                    _stage(0, r0t + (2 * q + 2) * SB, semA)
                if jj == 12:
                    _drain_stage(0, semA)
                if jj < LA:
                    _drain_row(b, gsems[b])
                else:
                    cps[jj].wait()

                @pl.loop(0, ECH // L)
                def _scale(gg):
                    v16 = lax.bitcast_convert_type(
                        vblk[slot, row, pl.ds(gg * L, L)], _f32)
                    for t in range(L):
                        spl = v16.at[jnp.full((L,), t, _i32)].get(
                            mode="promise_in_bounds")
                        rows4[b, gg * L + t] = rows4[b, gg * L + t] * spl

                pltpu.async_copy(rows4.at[b],
                                 acc.at[dblk.at[slot, row]],
                                 ssems[b], add=True)

                fj = jj + LA
                fb = fj % NBUF
                _drain_row(fb, ssems[fb])
                if fj < 16:
                    cps[fj] = pltpu.async_copy(
                        tab.at[sblk.at[fj // SB, fj % SB]],
                        rows4.at[fb], gsems[fb])
                else:
                    pltpu.async_copy(
                        tab.at[sblk.at[0, fj - 16]],
                        rows4.at[fb], gsems[fb])
            _stage(1, r0t + (2 * q + 3) * SB, semB)

        for b in range(LA):
            _drain_row(b, gsems[b])
        for b in range(LA, NBUF):
            _drain_row(b, ssems[b])
        _drain_stage(1, semB)

        plsc.subcore_barrier()
        pltpu.sync_copy(acc.at[pl.ds(row0, RPT)],
                        layers.at[k, c, pl.ds(row0, RPT)])
        if k < N_LAYERS - 1:
            @pl.loop(0, RPT // ZR)
            def _zero(zz):
                pltpu.sync_copy(zbuf, acc.at[pl.ds(row0 + zz * ZR, ZR)])
        plsc.subcore_barrier()

    tabs = [emb0.at[c]] + [layers.at[kk, c] for kk in range(N_LAYERS)]

    for idx_in, out_ref in ((u_idx, u_g), (p_idx, p_g)):
        @pl.loop(0, UPT // PPW)
        def _u_chunk(h):
            ub = s * UPT + h * PPW
            pltpu.sync_copy(idx_in.at[pl.ds(ub, PPW)], g32)
            cps = [pltpu.async_copy(tabs[b].at[g32], r64q.at[b], gsems[b])
                   for b in range(NT)]
            for cp in cps:
                cp.wait()

            @pl.loop(0, PPW)
            def _sum_u(i):
                r64q[0, i] = ((r64q[0, i] + r64q[1, i])
                              + (r64q[2, i] + r64q[3, i]))

            pltpu.sync_copy(r64q.at[0], out_ref.at[c, pl.ds(ub, PPW)])

    @pl.loop(0, NPT // ECH)
    def _neg_chunk(q):
        nb = s * NPT + q * ECH
        pltpu.sync_copy(n_idx.at[pl.ds(nb, ECH)], g128)
        cps = [pltpu.async_copy(tabs[b].at[g128], rows4.at[b], gsems[b])
               for b in range(NT)]
        for cp in cps:
            cp.wait()

        @pl.loop(0, ECH)
        def _sum_n(i):
            rows4[0, i] = ((rows4[0, i] + rows4[1, i])
                           + (rows4[2, i] + rows4[3, i]))

        pltpu.sync_copy(rows4.at[0], n_g.at[c, pl.ds(nb, ECH)])

    pltpu.sync_copy(up_idx.at[pl.ds(w * PPW, PPW)], g32)
    pltpu.async_copy(pop_u.at[g32], r32, g0).wait()
    pltpu.sync_copy(r32, up_g.at[pl.ds(w * PPW, PPW)])

    pltpu.sync_copy(pp_idx.at[pl.ds(w * PPW, PPW)], g32)
    pltpu.async_copy(pop_i.at[g32], r32, g0).wait()
    pltpu.sync_copy(r32, pp_g.at[pl.ds(w * PPW, PPW)])

    @pl.loop(0, NPW // PPW)
    def _npop_chunk(q):
        nb = w * NPW + q * PPW
        pltpu.sync_copy(np_idx.at[pl.ds(nb, PPW)], g32)
        pltpu.async_copy(pop_i.at[g32], rows32, g0).wait()
        pltpu.sync_copy(rows32, np_g.at[pl.ds(nb, PPW)])


_sc_call = functools.partial(
    pl.kernel,
    out_type=[
        jax.ShapeDtypeStruct((N_LAYERS, NC, N_PAD, H), _f32),
        jax.ShapeDtypeStruct((NC, B, H), _f32),
        jax.ShapeDtypeStruct((NC, B, H), _f32),
        jax.ShapeDtypeStruct((NC, BN, H), _f32),
        jax.ShapeDtypeStruct((B, EMB), _f32),
        jax.ShapeDtypeStruct((B, EMB), _f32),
        jax.ShapeDtypeStruct((BN, EMB), _f32),
    ],
    mesh=plsc.VectorSubcoreMesh(core_axis_name="c", subcore_axis_name="s",
                                num_cores=NC, num_subcores=NS),
    scratch_types=[
        pltpu.VMEM((2, SB, ECH), _i32),
        pltpu.VMEM((2, SB, ECH), _i32),
        pltpu.VMEM((2, SB, ECH), _i32),
        pltpu.VMEM((NBUF, ECH, H), _f32),
        pltpu.VMEM((ZR, L), _f32),
        pltpu.VMEM((NT, PPW, H), _f32),
        pltpu.VMEM((PPW,), _i32),
        pltpu.VMEM((PPW, EMB), _f32),
        pltpu.VMEM((ECH,), _i32),
        pltpu.VMEM((PPW, EMB), _f32),
        pltpu.VMEM_SHARED((N_PAD, H), _f32),
    ] + [pltpu.SemaphoreType.DMA] * 18,
    compiler_params=pltpu.CompilerParams(use_tc_tiling_on_sc=False),
)(_sc_body)


BB = 128
NSTEP = B // BB


def _tc_score_body(u_ref, pi_ref, ni_ref, up_ref, pip_ref, nip_ref,
                   o1, o2, o3, o4, o5, accs):
    i = pl.program_id(0)

    @pl.when(i == 0)
    def _init():
        for t in range(4):
            accs[t] = 0.0

    u0 = u_ref[0] * 0.25
    u1 = u_ref[1] * 0.25
    pi0 = pi_ref[0] * 0.25
    pi1 = pi_ref[1] * 0.25
    ni0 = ni_ref[0] * 0.25
    ni1 = ni_ref[1] * 0.25
    up = up_ref[...]
    pip = pip_ref[...]
    nip = nip_ref[...]

    nir0 = ni0.reshape(BB, NEG, H)
    nir1 = ni1.reshape(BB, NEG, H)
    nipr = nip.reshape(BB, NEG, EMB)

    un = jnp.sqrt(jnp.sum(u0 * u0, axis=1) + jnp.sum(u1 * u1, axis=1))
    upn = jnp.sqrt(jnp.sum(up * up, axis=1))
    pin = jnp.sqrt(jnp.sum(pi0 * pi0, axis=1) + jnp.sum(pi1 * pi1, axis=1))
    pipn = jnp.sqrt(jnp.sum(pip * pip, axis=1))
    nin = jnp.sqrt(jnp.sum(nir0 * nir0, axis=2) +
                   jnp.sum(nir1 * nir1, axis=2))
    nipn = jnp.sqrt(jnp.sum(nipr * nipr, axis=2))

    pos_prod = (jnp.sum(u0 * pi0, axis=1) + jnp.sum(u1 * pi1, axis=1))
    pos_pop_prod = jnp.sum(up * pip, axis=1)
    neg_prod = (jnp.sum(nir0 * u0[:, None, :], axis=2) +
                jnp.sum(nir1 * u1[:, None, :], axis=2))
    neg_pop_prod = jnp.sum(nipr * up[:, None, :], axis=2)

    pos_score = pos_prod / un / pin
    neg_score = neg_prod / un[:, None] / nin
    pos_pop_score = pos_pop_prod / upn / pipn / TAU2
    neg_pop_score = neg_pop_prod / upn[:, None] / nipn / TAU2

    neg_pop_exp = jnp.sum(jnp.exp(neg_pop_score), axis=1)
    pos_pop_exp = jnp.exp(pos_pop_score)
    l2 = jnp.sum(-jnp.log(pos_pop_exp / (pos_pop_exp + neg_pop_exp)))

    sig_pos = 1.0 / (1.0 + jnp.exp(-pos_pop_prod))
    sig_neg = 1.0 / (1.0 + jnp.exp(-neg_pop_prod))
    wpos = pos_score * sig_pos / TAU1
    wneg = neg_score * sig_neg / TAU1
    nexp = jnp.sum(jnp.exp(wneg), axis=1)
    pexp = jnp.exp(wpos)
    l1 = jnp.sum(-jnp.log(pexp / (pexp + nexp)))

    r1 = 0.5 * (jnp.sum(u0 * u0) + jnp.sum(u1 * u1) +
                jnp.sum(pi0 * pi0) + jnp.sum(pi1 * pi1) +
                jnp.sum(ni0 * ni0) + jnp.sum(ni1 * ni1))
    r2 = 0.5 * (jnp.sum(up * up) + jnp.sum(pip * pip) + jnp.sum(nip * nip))

    accs[0] += l1
    accs[1] += l2
    accs[2] += r1
    accs[3] += r2

    @pl.when(i == NSTEP - 1)
    def _fin():
        o1[0, 0] = (1.0 - W_LAMBDA) * accs[0] / B
        o2[0, 0] = W_LAMBDA * accs[1] / B
        o3[0, 0] = DECAY * (accs[2] + accs[3]) / B
        o4[0, 0] = DECAY * accs[3] / B
        o5[0, 0] = DECAY * accs[2] / B


_tc_score = pl.pallas_call(
    _tc_score_body,
    grid=(NSTEP,),
    in_specs=[
        pl.BlockSpec((NC, BB, H), lambda i: (0, i, 0)),
        pl.BlockSpec((NC, BB, H), lambda i: (0, i, 0)),
        pl.BlockSpec((NC, BB * NEG, H), lambda i: (0, i, 0)),
        pl.BlockSpec((BB, EMB), lambda i: (i, 0)),
        pl.BlockSpec((BB, EMB), lambda i: (i, 0)),
        pl.BlockSpec((BB * NEG, EMB), lambda i: (i, 0)),
    ],
    out_shape=[jax.ShapeDtypeStruct((1, 1), _f32)] * 5,
    out_specs=[pl.BlockSpec(memory_space=pltpu.SMEM)] * 5,
    scratch_shapes=[pltpu.SMEM((4,), _f32)],
)


def kernel(users, pos_items, neg_items, users_pop, pos_items_pop,
           neg_items_pop, embed_user, embed_item, embed_user_pop,
           embed_item_pop, graph_src, graph_dst, graph_val):
    ae = jnp.concatenate([embed_user, embed_item], axis=0)
    ae = jnp.pad(ae, ((0, N_PAD - N), (0, 0)))
    emb0 = jnp.stack([ae[:, :H], ae[:, H:]], axis=0)

    pad = SROWS_P * ECH - E
    srcs = jnp.concatenate(
        [graph_src.astype(_i32),
         jnp.zeros((pad,), _i32)]).reshape(SROWS_P, ECH)
    dsts = jnp.concatenate(
        [graph_dst.astype(_i32),
         jnp.zeros((pad,), _i32)]).reshape(SROWS_P, ECH)
    vbits = jnp.concatenate(
        [lax.bitcast_convert_type(graph_val.astype(_f32), _i32),
         jnp.zeros((pad,), _i32)]).reshape(SROWS_P, ECH)

    u_idx = users.astype(_i32)
    p_idx = pos_items.astype(_i32) + N_USERS
    n_idx = neg_items.astype(_i32) + N_USERS
    up_idx = users_pop.astype(_i32)
    pp_idx = pos_items_pop.astype(_i32)
    np_idx = neg_items_pop.astype(_i32)

    (_, u_g, p_g, n_g, up_g, pp_g, np_g) = _sc_call(
        emb0, srcs, dsts, vbits, u_idx, p_idx, n_idx, up_idx, pp_idx,
        np_idx, embed_user_pop, embed_item_pop)

    o1, o2, o3, o4, o5 = _tc_score(u_g, p_g, n_g, up_g, pp_g, np_g)
    return (o1[0, 0], o2[0, 0], o3[0, 0], o4[0, 0], o5[0, 0])


# speedup vs baseline: 1.0464x; 1.0464x over previous
import functools

import jax
import jax.numpy as jnp
from jax import lax
from jax.experimental import pallas as pl
from jax.experimental.pallas import tpu as pltpu
from jax.experimental.pallas import tpu_sc as plsc

N_USERS = 50000
N_ITEMS = 50000
N = N_USERS + N_ITEMS
EMB = 32
H = 16
N_LAYERS = 3
NT = N_LAYERS + 1
B = 1024
NEG = 64
BN = B * NEG
TAU1 = 0.1
TAU2 = 0.1
W_LAMBDA = 0.5
DECAY = 1e-4
E = 800000

NC, NS, L = 2, 16, 16
NW = NC * NS

N_PAD = 102400
RPT = N_PAD // NS
ECH = 128
SB = 8
NBLK = 50
NIT = NBLK // 2
EPT = NBLK * SB * ECH
E_PAD = EPT * NS
SROWS = E_PAD // ECH
SROWS_P = SROWS + 2 * SB
NBUF = 8
LA = 5
ZR = 64
UPT = B // NS
NPT = BN // NS
PPW = B // NW
NPW = BN // NW

_f32 = jnp.float32
_i32 = jnp.int32


def _sc_body(emb0, e_s, e_d, e_v, u_idx, p_idx, n_idx, up_idx, pp_idx,
             np_idx, pop_u, pop_i,
             layers, u_g, p_g, n_g, up_g, pp_g, np_g,
             sblk, dblk, vblk, rows4, zbuf, r64q, g32, r32, g128, rows32,
             acc, g0, g1, g2, g3, g4, g5, g6, g7,
             s0, s1, s2, s3, s4, s5, s6, s7, semA, semB):
    c = lax.axis_index("c")
    s = lax.axis_index("s")
    w = s * NC + c
    row0 = s * RPT
    gsems = [g0, g1, g2, g3, g4, g5, g6, g7]
    ssems = [s0, s1, s2, s3, s4, s5, s6, s7]

    def _drain_row(b, sem):
        pltpu.make_async_copy(
            emb0.at[0, pl.ds(0, ECH)], rows4.at[b], sem).wait()

    def _stage(slot, r, sem):
        pltpu.async_copy(e_s.at[pl.ds(r, SB)], sblk.at[slot], sem)
        pltpu.async_copy(e_d.at[pl.ds(r, SB)], dblk.at[slot], sem)
        pltpu.async_copy(e_v.at[pl.ds(r, SB)], vblk.at[slot], sem)

    def _drain_stage(slot, sem):
        pltpu.make_async_copy(e_s.at[pl.ds(0, SB)], sblk.at[slot], sem).wait()
        pltpu.make_async_copy(e_d.at[pl.ds(0, SB)], dblk.at[slot], sem).wait()
        pltpu.make_async_copy(e_v.at[pl.ds(0, SB)], vblk.at[slot], sem).wait()

    @pl.loop(0, ZR)
    def _zfill(i):
        zbuf[i] = jnp.zeros((L,), _f32)

    @pl.loop(0, RPT // ZR)
    def _zero0(zz):
        pltpu.sync_copy(zbuf, acc.at[pl.ds(row0 + zz * ZR, ZR)])
    plsc.subcore_barrier()

    r0t = s * (NBLK * SB)

    for k in range(N_LAYERS):
        tab = emb0.at[c] if k == 0 else layers.at[k - 1, c]

        pltpu.sync_copy(e_s.at[pl.ds(r0t, SB)], sblk.at[0])
        pltpu.sync_copy(e_d.at[pl.ds(r0t, SB)], dblk.at[0])
        pltpu.sync_copy(e_v.at[pl.ds(r0t, SB)], vblk.at[0])
        _stage(1, r0t + SB, semB)

        @pl.loop(0, ECH)
        def _zb(i):
            for b in range(LA, NBUF):
                rows4[b, i] = jnp.zeros((L,), _f32)
        for b in range(LA, NBUF):
            pltpu.async_copy(rows4.at[b], acc.at[dblk.at[0, 0]],
                             ssems[b], add=True)
        for b in range(LA):
            pltpu.async_copy(tab.at[sblk.at[0, b]], rows4.at[b],
                             gsems[b])

        @pl.loop(0, NIT)
        def _edge_iter(q):
            cps = [None] * 16
            for jj in range(16):
                slot, row, b = jj // SB, jj % SB, jj % NBUF
                if jj == 5:
                    _drain_stage(1, semB)
                if jj == 8:
                    _stage(0, r0t + (2 * q + 2) * SB, semA)
                if jj == 12:
                    _drain_stage(0, semA)
                if jj < LA:
                    _drain_row(b, gsems[b])
                else:
                    cps[jj].wait()

                @pl.loop(0, ECH // L)
                def _scale(gg):
                    v16 = lax.bitcast_convert_type(
                        vblk[slot, row, pl.ds(gg * L, L)], _f32)
                    for t in range(L):
                        spl = v16.at[jnp.full((L,), t, _i32)].get(
                            mode="promise_in_bounds")
                        rows4[b, gg * L + t] = rows4[b, gg * L + t] * spl

                pltpu.async_copy(rows4.at[b],
                                 acc.at[dblk.at[slot, row]],
                                 ssems[b], add=True)

                fj = jj + LA
                fb = fj % NBUF
                _drain_row(fb, ssems[fb])
                if fj < 16:
                    cps[fj] = pltpu.async_copy(
                        tab.at[sblk.at[fj // SB, fj % SB]],
                        rows4.at[fb], gsems[fb])
                else:
                    pltpu.async_copy(
                        tab.at[sblk.at[0, fj - 16]],
                        rows4.at[fb], gsems[fb])
            _stage(1, r0t + (2 * q + 3) * SB, semB)

        for b in range(LA):
            _drain_row(b, gsems[b])
        for b in range(LA, NBUF):
            _drain_row(b, ssems[b])
        _drain_stage(1, semB)

        plsc.subcore_barrier()
        pltpu.sync_copy(acc.at[pl.ds(row0, RPT)],
                        layers.at[k, c, pl.ds(row0, RPT)])
        if k < N_LAYERS - 1:
            @pl.loop(0, RPT // ZR)
            def _zero(zz):
                pltpu.sync_copy(zbuf, acc.at[pl.ds(row0 + zz * ZR, ZR)])
        plsc.subcore_barrier()

    tabs = [emb0.at[c]] + [layers.at[kk, c] for kk in range(N_LAYERS)]

    for idx_in, out_ref in ((u_idx, u_g), (p_idx, p_g)):
        @pl.loop(0, UPT // PPW)
        def _u_chunk(h):
            ub = s * UPT + h * PPW
            pltpu.sync_copy(idx_in.at[pl.ds(ub, PPW)], g32)
            cps = [pltpu.async_copy(tabs[b].at[g32], r64q.at[b], gsems[b])
                   for b in range(NT)]
            for cp in cps:
                cp.wait()

            @pl.loop(0, PPW)
            def _sum_u(i):
                r64q[0, i] = ((r64q[0, i] + r64q[1, i])
                              + (r64q[2, i] + r64q[3, i]))

            pltpu.sync_copy(r64q.at[0], out_ref.at[c, pl.ds(ub, PPW)])

    @pl.loop(0, NPT // ECH)
    def _neg_chunk(q):
        nb = s * NPT + q * ECH
        pltpu.sync_copy(n_idx.at[pl.ds(nb, ECH)], g128)
        cps = [pltpu.async_copy(tabs[b].at[g128], rows4.at[b], gsems[b])
               for b in range(NT)]
        for cp in cps:
            cp.wait()

        @pl.loop(0, ECH)
        def _sum_n(i):
            rows4[0, i] = ((rows4[0, i] + rows4[1, i])
                           + (rows4[2, i] + rows4[3, i]))

        pltpu.sync_copy(rows4.at[0], n_g.at[c, pl.ds(nb, ECH)])

    pltpu.sync_copy(up_idx.at[pl.ds(w * PPW, PPW)], g32)
    pltpu.async_copy(pop_u.at[g32], r32, g0).wait()
    pltpu.sync_copy(r32, up_g.at[pl.ds(w * PPW, PPW)])

    pltpu.sync_copy(pp_idx.at[pl.ds(w * PPW, PPW)], g32)
    pltpu.async_copy(pop_i.at[g32], r32, g0).wait()
    pltpu.sync_copy(r32, pp_g.at[pl.ds(w * PPW, PPW)])

    @pl.loop(0, NPW // PPW)
    def _npop_chunk(q):
        nb = w * NPW + q * PPW
        pltpu.sync_copy(np_idx.at[pl.ds(nb, PPW)], g32)
        pltpu.async_copy(pop_i.at[g32], rows32, g0).wait()
        pltpu.sync_copy(rows32, np_g.at[pl.ds(nb, PPW)])


_sc_call = functools.partial(
    pl.kernel,
    out_type=[
        jax.ShapeDtypeStruct((N_LAYERS, NC, N_PAD, H), _f32),
        jax.ShapeDtypeStruct((NC, B, H), _f32),
        jax.ShapeDtypeStruct((NC, B, H), _f32),
        jax.ShapeDtypeStruct((NC, BN, H), _f32),
        jax.ShapeDtypeStruct((B, EMB), _f32),
        jax.ShapeDtypeStruct((B, EMB), _f32),
        jax.ShapeDtypeStruct((BN, EMB), _f32),
    ],
    mesh=plsc.VectorSubcoreMesh(core_axis_name="c", subcore_axis_name="s",
                                num_cores=NC, num_subcores=NS),
    scratch_types=[
        pltpu.VMEM((2, SB, ECH), _i32),
        pltpu.VMEM((2, SB, ECH), _i32),
        pltpu.VMEM((2, SB, ECH), _i32),
        pltpu.VMEM((NBUF, ECH, H), _f32),
        pltpu.VMEM((ZR, L), _f32),
        pltpu.VMEM((NT, PPW, H), _f32),
        pltpu.VMEM((PPW,), _i32),
        pltpu.VMEM((PPW, EMB), _f32),
        pltpu.VMEM((ECH,), _i32),
        pltpu.VMEM((PPW, EMB), _f32),
        pltpu.VMEM_SHARED((N_PAD, H), _f32),
    ] + [pltpu.SemaphoreType.DMA] * 18,
    compiler_params=pltpu.CompilerParams(use_tc_tiling_on_sc=False),
)(_sc_body)


BB = 128
NSTEP = B // BB


def _tc_score_body(u_ref, pi_ref, ni_ref, up_ref, pip_ref, nip_ref,
                   o1, o2, o3, o4, o5, accs):
    i = pl.program_id(0)

    @pl.when(i == 0)
    def _init():
        for t in range(4):
            accs[t] = 0.0

    u = u_ref[...] * 0.25
    pi = pi_ref[...] * 0.25
    ni = ni_ref[...] * 0.25
    up = up_ref[...]
    pip = pip_ref[...]
    nip = nip_ref[...]

    nir = ni.reshape(BB, NEG, EMB)
    nipr = nip.reshape(BB, NEG, EMB)

    un = jnp.sqrt(jnp.sum(u * u, axis=1))
    upn = jnp.sqrt(jnp.sum(up * up, axis=1))
    pin = jnp.sqrt(jnp.sum(pi * pi, axis=1))
    pipn = jnp.sqrt(jnp.sum(pip * pip, axis=1))
    nin = jnp.sqrt(jnp.sum(nir * nir, axis=2))
    nipn = jnp.sqrt(jnp.sum(nipr * nipr, axis=2))

    pos_prod = jnp.sum(u * pi, axis=1)
    pos_pop_prod = jnp.sum(up * pip, axis=1)
    neg_prod = jnp.sum(nir * u[:, None, :], axis=2)
    neg_pop_prod = jnp.sum(nipr * up[:, None, :], axis=2)

    pos_score = pos_prod / un / pin
    neg_score = neg_prod / un[:, None] / nin
    pos_pop_score = pos_pop_prod / upn / pipn / TAU2
    neg_pop_score = neg_pop_prod / upn[:, None] / nipn / TAU2

    neg_pop_exp = jnp.sum(jnp.exp(neg_pop_score), axis=1)
    pos_pop_exp = jnp.exp(pos_pop_score)
    l2 = jnp.sum(-jnp.log(pos_pop_exp / (pos_pop_exp + neg_pop_exp)))

    sig_pos = 1.0 / (1.0 + jnp.exp(-pos_pop_prod))
    sig_neg = 1.0 / (1.0 + jnp.exp(-neg_pop_prod))
    wpos = pos_score * sig_pos / TAU1
    wneg = neg_score * sig_neg / TAU1
    nexp = jnp.sum(jnp.exp(wneg), axis=1)
    pexp = jnp.exp(wpos)
    l1 = jnp.sum(-jnp.log(pexp / (pexp + nexp)))

    r1 = 0.5 * (jnp.sum(u * u) + jnp.sum(pi * pi) + jnp.sum(ni * ni))
    r2 = 0.5 * (jnp.sum(up * up) + jnp.sum(pip * pip) + jnp.sum(nip * nip))

    accs[0] += l1
    accs[1] += l2
    accs[2] += r1
    accs[3] += r2

    @pl.when(i == NSTEP - 1)
    def _fin():
        o1[0, 0] = (1.0 - W_LAMBDA) * accs[0] / B
        o2[0, 0] = W_LAMBDA * accs[1] / B
        o3[0, 0] = DECAY * (accs[2] + accs[3]) / B
        o4[0, 0] = DECAY * accs[3] / B
        o5[0, 0] = DECAY * accs[2] / B


_tc_score = pl.pallas_call(
    _tc_score_body,
    grid=(NSTEP,),
    in_specs=[
        pl.BlockSpec((BB, EMB), lambda i: (i, 0)),
        pl.BlockSpec((BB, EMB), lambda i: (i, 0)),
        pl.BlockSpec((BB * NEG, EMB), lambda i: (i, 0)),
        pl.BlockSpec((BB, EMB), lambda i: (i, 0)),
        pl.BlockSpec((BB, EMB), lambda i: (i, 0)),
        pl.BlockSpec((BB * NEG, EMB), lambda i: (i, 0)),
    ],
    out_shape=[jax.ShapeDtypeStruct((1, 1), _f32)] * 5,
    out_specs=[pl.BlockSpec(memory_space=pltpu.SMEM)] * 5,
    scratch_shapes=[pltpu.SMEM((4,), _f32)],
)


def kernel(users, pos_items, neg_items, users_pop, pos_items_pop,
           neg_items_pop, embed_user, embed_item, embed_user_pop,
           embed_item_pop, graph_src, graph_dst, graph_val):
    ae = jnp.concatenate([embed_user, embed_item], axis=0)
    ae = jnp.pad(ae, ((0, N_PAD - N), (0, 0)))
    emb0 = jnp.stack([ae[:, :H], ae[:, H:]], axis=0)

    pad = SROWS_P * ECH - E
    srcs = jnp.concatenate(
        [graph_src.astype(_i32),
         jnp.zeros((pad,), _i32)]).reshape(SROWS_P, ECH)
    dsts = jnp.concatenate(
        [graph_dst.astype(_i32),
         jnp.zeros((pad,), _i32)]).reshape(SROWS_P, ECH)
    vbits = jnp.concatenate(
        [lax.bitcast_convert_type(graph_val.astype(_f32), _i32),
         jnp.zeros((pad,), _i32)]).reshape(SROWS_P, ECH)

    u_idx = users.astype(_i32)
    p_idx = pos_items.astype(_i32) + N_USERS
    n_idx = neg_items.astype(_i32) + N_USERS
    up_idx = users_pop.astype(_i32)
    pp_idx = pos_items_pop.astype(_i32)
    np_idx = neg_items_pop.astype(_i32)

    (_, u_g, p_g, n_g, up_g, pp_g, np_g) = _sc_call(
        emb0, srcs, dsts, vbits, u_idx, p_idx, n_idx, up_idx, pp_idx,
        np_idx, embed_user_pop, embed_item_pop)

    u = jnp.transpose(u_g, (1, 0, 2)).reshape(B, EMB)
    pi = jnp.transpose(p_g, (1, 0, 2)).reshape(B, EMB)
    ni = jnp.transpose(n_g, (1, 0, 2)).reshape(BN, EMB)

    o1, o2, o3, o4, o5 = _tc_score(u, pi, ni, up_g, pp_g, np_g)
    return (o1[0, 0], o2[0, 0], o3[0, 0], o4[0, 0], o5[0, 0])
